# baseline (speedup 1.0000x reference)
import dataclasses
import functools

import jax
import jax.numpy as jnp
from jax import lax
from jax.experimental import pallas as pl
from jax.experimental.pallas import tpu as pltpu
from jax.experimental.pallas import tpu_sc as plsc

H, W = 1024, 2048
HW = H * W
BR = 128
CH = 128
NCH = H // CH


def _coords(se0, se1):
    gx = 2.0 * ((se0 * 1024.0) / 2047.0 - 0.5)
    gy = 2.0 * ((se1 * 1024.0) / 1023.0 - 0.5)
    x = ((gx + 1.0) * 2048.0) / 2.0 - 0.5
    y = ((gy + 1.0) * 1024.0) / 2.0 - 0.5
    x0 = jnp.floor(x)
    y0 = jnp.floor(y)
    return x, y, x0, y0


def _clip_idx(xi, yi):
    xc = jnp.clip(xi, 0, W - 1).astype(jnp.int32)
    yc = jnp.clip(yi, 0, H - 1).astype(jnp.int32)
    return yc * W + xc


def _valid(xi, yi):
    return (xi >= 0) & (xi <= W - 1) & (yi >= 0) & (yi <= H - 1)


def _a_kernel(p0_r, p1_r, p5_r, p6_r, xm_r, ym_r, seed_o, se0_o, se1_o, idx_o):
    o0 = jnp.tanh(p0_r[...])
    o1 = jnp.tanh(p1_r[...])
    se0 = o0 + xm_r[...]
    se1 = o1 + ym_r[...]
    se0_o[...] = se0
    se1_o[...] = se1
    mx = jnp.maximum(p5_r[...], p6_r[...])
    e0 = jnp.exp(p5_r[...] - mx)
    e1 = jnp.exp(p6_r[...] - mx)
    seed_o[...] = e1 / (e0 + e1)
    x, y, x0, y0 = _coords(se0, se1)
    x1 = x0 + 1.0
    y1 = y0 + 1.0
    xc0 = jnp.clip(x0, 0, W - 1).astype(jnp.int32)
    xc1 = jnp.clip(x1, 0, W - 1).astype(jnp.int32)
    yc0 = jnp.clip(y0, 0, H - 1).astype(jnp.int32)
    yc1 = jnp.clip(y1, 0, H - 1).astype(jnp.int32)
    lin00 = yc0 * W + xc0
    lin20 = yc1 * W + xc0
    o0 = xc0 & 31
    idx_o[0] = lin00 >> 5
    idx_o[1] = lin20 >> 5
    idx_o[2] = o0
    idx_o[3] = o0 + (xc1 - xc0)


def _b_kernel(se0_r, se1_r, g0_r, g1_r, e0_o, e1_o):
    se0 = se0_r[...]
    se1 = se1_r[...]
    x, y, x0, y0 = _coords(se0, se1)
    x1 = x0 + 1.0
    y1 = y0 + 1.0
    wx1 = x - x0
    wx0 = 1.0 - wx1
    wy1 = y - y0
    wy0 = 1.0 - wy1
    ws = (wx0 * wy0, wx1 * wy0, wx0 * wy1, wx1 * wy1)
    vs = (_valid(x0, y0), _valid(x1, y0), _valid(x0, y1), _valid(x1, y1))
    acc0 = jnp.zeros_like(se0)
    acc1 = jnp.zeros_like(se1)
    for t in range(4):
        o0t = jnp.where(vs[t], jnp.tanh(g0_r[t]), 0.0)
        o1t = jnp.where(vs[t], jnp.tanh(g1_r[t]), 0.0)
        if t == 0:
            acc0 = o0t * ws[t]
            acc1 = o1t * ws[t]
        else:
            acc0 = acc0 + o0t * ws[t]
            acc1 = acc1 + o1t * ws[t]
    e0_o[...] = se0 + acc0
    e1_o[...] = se1 + acc1


def _c_kernel(seed_r, e0_r, e1_r, s0_r, s1_r, inst_o, uncl_s, prop_s):
    colid = jax.lax.broadcasted_iota(jnp.int32, (CH, W), 1)
    rowid = jax.lax.broadcasted_iota(jnp.int32, (CH, W), 0)

    def init_chunk(i, n):
        sl = pl.ds(i * CH, CH)
        mk = (seed_r[sl, :] > 0.5).astype(jnp.uint8)
        uncl_s[sl, :] = mk
        inst_o[sl, :] = jnp.zeros((CH, W), jnp.uint8)
        return n + jnp.sum(mk.astype(jnp.int32))

    n0 = jax.lax.fori_loop(0, NCH, init_chunk, jnp.int32(0))

    def body(carry):
        count, _un = carry

        def p12(i, c):
            m, idx = c
            sl = pl.ds(i * CH, CH)
            sc = jnp.where(uncl_s[sl, :].astype(jnp.int32) > 0,
                           seed_r[sl, :], 0.0)
            cm = jnp.max(sc)
            flat = (rowid + i * CH) * W + colid
            cidx = jnp.min(jnp.where(sc == cm, flat, HW))
            idx_new = jnp.where(cm > m, cidx, idx)
            return (jnp.maximum(m, cm), idx_new)

        _m, idx = jax.lax.fori_loop(0, NCH, p12, (jnp.float32(0.0), jnp.int32(HW)))
        r = idx // W
        c = idx % W

        def _gather(ref):
            row = ref[pl.ds(r, 1), :]
            return jnp.sum(jnp.where(colid[0:1, :] == c, row, 0.0))

        c0 = _gather(e0_r)
        c1 = _gather(e1_r)
        s0 = jnp.exp(_gather(s0_r) * 10.0)
        s1 = jnp.exp(_gather(s1_r) * 10.0)

        def p3(i, acc):
            psum, usum = acc
            sl = pl.ds(i * CH, CH)
            d0 = e0_r[sl, :] - c0
            d1 = e1_r[sl, :] - c1
            q = d0 * d0 * s0 + d1 * d1 * s1
            dist = jnp.exp(-1.0 * q)
            pr_i = ((dist > 0.5) & (seed_r[sl, :] > 0.5)).astype(jnp.int32)
            flat = (rowid + i * CH) * W + colid
            unc_i = jnp.where(flat == idx, 0,
                              uncl_s[sl, :].astype(jnp.int32))
            uncl_s[sl, :] = unc_i.astype(jnp.uint8)
            prop_s[sl, :] = pr_i.astype(jnp.uint8)
            psum = psum + jnp.sum(pr_i)
            usum = usum + jnp.sum(pr_i * unc_i)
            return (psum, usum)

        psum, usum = jax.lax.fori_loop(0, NCH, p3, (jnp.int32(0), jnp.int32(0)))
        ratio_ok = (usum.astype(jnp.float32)
                    / jnp.maximum(psum, 1).astype(jnp.float32)) > 0.5
        accept = (psum > 160) & ratio_ok
        acc_i = jnp.where(accept, jnp.int32(1), jnp.int32(0))
        lab_i = count & 255

        def p4(i, un):
            sl = pl.ds(i * CH, CH)
            pr_i = prop_s[sl, :].astype(jnp.int32)
            inst_i = inst_o[sl, :].astype(jnp.int32)
            inst_o[sl, :] = jnp.where(pr_i * acc_i > 0, lab_i,
                                      inst_i).astype(jnp.uint8)
            unc_i = jnp.where(pr_i > 0, 0, uncl_s[sl, :].astype(jnp.int32))
            uncl_s[sl, :] = unc_i.astype(jnp.uint8)
            return un + jnp.sum(unc_i)

        un_new = jax.lax.fori_loop(0, NCH, p4, jnp.int32(0))
        count_new = count + jnp.where(accept, jnp.int32(1), jnp.int32(0))
        return (count_new, un_new)

    count_fin, _ = jax.lax.while_loop(lambda cr: cr[1] > 160, body,
                                      (jnp.int32(1), n0))

    def rem(l, z):
        li = l & 255

        def cnt_chunk(i, n):
            sl = pl.ds(i * CH, CH)
            return n + jnp.sum(
                (inst_o[sl, :].astype(jnp.int32) == li).astype(jnp.int32))

        n = jax.lax.fori_loop(0, NCH, cnt_chunk, jnp.int32(0))

        @pl.when(n < 160)
        def _():
            def rm(i, zz):
                sl = pl.ds(i * CH, CH)
                inst_i = inst_o[sl, :].astype(jnp.int32)
                inst_o[sl, :] = jnp.where(inst_i == li, 0,
                                          inst_i).astype(jnp.uint8)
                return zz

            jax.lax.fori_loop(0, NCH, rm, jnp.int32(0))

        return z

    jax.lax.fori_loop(1, count_fin, rem, jnp.int32(0))


def _stage_a(p0, p1, p5, p6, xm, ym, interpret=False):
    f32 = jnp.float32
    return pl.pallas_call(
        _a_kernel,
        grid=(H // BR,),
        in_specs=[
            pl.BlockSpec((BR, W), lambda i: (i, 0)),
            pl.BlockSpec((BR, W), lambda i: (i, 0)),
            pl.BlockSpec((BR, W), lambda i: (i, 0)),
            pl.BlockSpec((BR, W), lambda i: (i, 0)),
            pl.BlockSpec((1, W), lambda i: (0, 0)),
            pl.BlockSpec((BR, 1), lambda i: (i, 0)),
        ],
        out_specs=[
            pl.BlockSpec((BR, W), lambda i: (i, 0)),
            pl.BlockSpec((BR, W), lambda i: (i, 0)),
            pl.BlockSpec((BR, W), lambda i: (i, 0)),
            pl.BlockSpec((4, BR, W), lambda i: (0, i, 0)),
        ],
        out_shape=[
            jax.ShapeDtypeStruct((H, W), f32),
            jax.ShapeDtypeStruct((H, W), f32),
            jax.ShapeDtypeStruct((H, W), f32),
            jax.ShapeDtypeStruct((4, H, W), jnp.int32),
        ],
        interpret=interpret,
    )(p0, p1, p5, p6, xm, ym)


def _stage_b(se0, se1, g0, g1, interpret=False):
    f32 = jnp.float32
    return pl.pallas_call(
        _b_kernel,
        grid=(H // BR,),
        in_specs=[
            pl.BlockSpec((BR, W), lambda i: (i, 0)),
            pl.BlockSpec((BR, W), lambda i: (i, 0)),
            pl.BlockSpec((4, BR, W), lambda i: (0, i, 0)),
            pl.BlockSpec((4, BR, W), lambda i: (0, i, 0)),
        ],
        out_specs=[
            pl.BlockSpec((BR, W), lambda i: (i, 0)),
            pl.BlockSpec((BR, W), lambda i: (i, 0)),
        ],
        out_shape=[
            jax.ShapeDtypeStruct((H, W), f32),
            jax.ShapeDtypeStruct((H, W), f32),
        ],
        interpret=interpret,
    )(se0, se1, g0, g1)


def _stage_c(seed, e0, e1, s0, s1, interpret=False):
    return pl.pallas_call(
        _c_kernel,
        in_specs=[pl.BlockSpec(memory_space=pltpu.VMEM)] * 5,
        out_specs=pl.BlockSpec(memory_space=pltpu.VMEM),
        out_shape=jax.ShapeDtypeStruct((H, W), jnp.uint8),
        scratch_shapes=[
            pltpu.VMEM((H, W), jnp.uint8),
            pltpu.VMEM((H, W), jnp.uint8),
        ],
        compiler_params=pltpu.CompilerParams(
            vmem_limit_bytes=100 * 1024 * 1024,
        ),
        interpret=interpret,
    )(seed, e0, e1, s0, s1)


SC_NW = 32
SC_PW = HW // SC_NW
SC_CHK = 256
SC_NIT = SC_PW // SC_CHK
NROW = HW // 32


def _mk_table(p0, p1):
    z = jnp.zeros((1, 32), jnp.float32)
    a0 = p0.reshape(NROW, 32)
    b0 = p1.reshape(NROW, 32)
    a1 = jnp.concatenate([a0[1:], z], axis=0)
    b1 = jnp.concatenate([b0[1:], z], axis=0)
    return jnp.concatenate([a0, a1, b0, b1], axis=1)


def _sc_gather(table, r0, r1, o0, o1):
    mesh = plsc.VectorSubcoreMesh(core_axis_name="c", subcore_axis_name="s")
    cp = pltpu.CompilerParams()
    if "needs_layout_passes" in pltpu.CompilerParams.__dataclass_fields__:
        cp = dataclasses.replace(cp, needs_layout_passes=False)

    @functools.partial(
        pl.kernel,
        mesh=mesh,
        compiler_params=cp,
        out_type=[
            jax.ShapeDtypeStruct((4, HW), jnp.float32),
            jax.ShapeDtypeStruct((4, HW), jnp.float32),
        ],
        scratch_types=[
            pltpu.VMEM((SC_CHK,), jnp.int32),
            pltpu.VMEM((SC_CHK,), jnp.int32),
            pltpu.VMEM((SC_CHK,), jnp.int32),
            pltpu.VMEM((SC_CHK,), jnp.int32),
            pltpu.VMEM((SC_CHK, 128), jnp.float32),
            pltpu.VMEM((SC_CHK, 128), jnp.float32),
            pltpu.VMEM((8, SC_CHK), jnp.float32),
            pltpu.SemaphoreType.DMA,
            pltpu.SemaphoreType.DMA,
        ],
    )
    def k(t_hbm, r0_hbm, r1_hbm, o0_hbm, o1_hbm, g0_hbm, g1_hbm,
          i0_v, i1_v, l0_v, l1_v, rows0_v, rows1_v, out_v, sem0, sem1):
        wid = lax.axis_index("s") * 2 + lax.axis_index("c")
        base = wid * SC_PW

        @pl.loop(0, SC_NIT)
        def _(it):
            off = base + it * SC_CHK
            pltpu.sync_copy(r0_hbm.at[pl.ds(off, SC_CHK)], i0_v)
            pltpu.sync_copy(r1_hbm.at[pl.ds(off, SC_CHK)], i1_v)
            cp0 = pltpu.async_copy(t_hbm.at[i0_v], rows0_v, sem0)
            cp1 = pltpu.async_copy(t_hbm.at[i1_v], rows1_v, sem1)
            pltpu.sync_copy(o0_hbm.at[pl.ds(off, SC_CHK)], l0_v)
            pltpu.sync_copy(o1_hbm.at[pl.ds(off, SC_CHK)], l1_v)
            cp0.wait()
            cp1.wait()

            @pl.loop(0, SC_CHK, step=16)
            def _(cb):
                rvec = cb + lax.iota(jnp.int32, 16)
                l0 = l0_v[pl.ds(cb, 16)]
                l1 = l1_v[pl.ds(cb, 16)]
                out_v[0, pl.ds(cb, 16)] = plsc.load_gather(rows0_v, [rvec, l0])
                out_v[1, pl.ds(cb, 16)] = plsc.load_gather(rows0_v, [rvec, l1])
                out_v[2, pl.ds(cb, 16)] = plsc.load_gather(rows1_v, [rvec, l0])
                out_v[3, pl.ds(cb, 16)] = plsc.load_gather(rows1_v, [rvec, l1])
                out_v[4, pl.ds(cb, 16)] = plsc.load_gather(rows0_v,
                                                           [rvec, l0 + 64])
                out_v[5, pl.ds(cb, 16)] = plsc.load_gather(rows0_v,
                                                           [rvec, l1 + 64])
                out_v[6, pl.ds(cb, 16)] = plsc.load_gather(rows1_v,
                                                           [rvec, l0 + 64])
                out_v[7, pl.ds(cb, 16)] = plsc.load_gather(rows1_v,
                                                           [rvec, l1 + 64])

            pltpu.sync_copy(out_v.at[0], g0_hbm.at[0, pl.ds(off, SC_CHK)])
            pltpu.sync_copy(out_v.at[1], g0_hbm.at[1, pl.ds(off, SC_CHK)])
            pltpu.sync_copy(out_v.at[2], g0_hbm.at[2, pl.ds(off, SC_CHK)])
            pltpu.sync_copy(out_v.at[3], g0_hbm.at[3, pl.ds(off, SC_CHK)])
            pltpu.sync_copy(out_v.at[4], g1_hbm.at[0, pl.ds(off, SC_CHK)])
            pltpu.sync_copy(out_v.at[5], g1_hbm.at[1, pl.ds(off, SC_CHK)])
            pltpu.sync_copy(out_v.at[6], g1_hbm.at[2, pl.ds(off, SC_CHK)])
            pltpu.sync_copy(out_v.at[7], g1_hbm.at[3, pl.ds(off, SC_CHK)])

    return k(table, r0, r1, o0, o1)


def _pipeline(prediction, interpret=False):
    pred = prediction[0]
    p0, p1 = pred[0], pred[1]
    sg0, sg1 = pred[2], pred[3]
    p5, p6 = pred[5], pred[6]
    xm = jnp.linspace(0.0, 2.0, 2048).reshape(1, W)
    ym = jnp.linspace(0.0, 1.0, 1024).reshape(H, 1)
    seed, se0, se1, idx4 = _stage_a(p0, p1, p5, p6, xm, ym, interpret=interpret)
    table = _mk_table(p0, p1)
    r0 = idx4[0].reshape(-1)
    r1 = idx4[1].reshape(-1)
    o0 = idx4[2].reshape(-1)
    o1 = idx4[3].reshape(-1)
    if interpret:
        g0f = jnp.stack([table[r0, o0], table[r0, o1],
                         table[r1, o0], table[r1, o1]])
        g1f = jnp.stack([table[r0, o0 + 64], table[r0, o1 + 64],
                         table[r1, o0 + 64], table[r1, o1 + 64]])
    else:
        g0f, g1f = _sc_gather(table, r0, r1, o0, o1)
    g0 = g0f.reshape(4, H, W)
    g1 = g1f.reshape(4, H, W)
    e0, e1 = _stage_b(se0, se1, g0, g1, interpret=interpret)
    inst = _stage_c(seed, e0, e1, sg0, sg1, interpret=interpret)
    return inst.reshape(1, H, W)


def kernel(prediction):
    return _pipeline(prediction)


# speedup vs baseline: 5.6299x; 5.6299x over previous
import dataclasses
import functools

import jax
import jax.numpy as jnp
from jax import lax
from jax.experimental import pallas as pl
from jax.experimental.pallas import tpu as pltpu
from jax.experimental.pallas import tpu_sc as plsc

H, W = 1024, 2048
HW = H * W
BR = 128
CH = 128
NCH = H // CH


def _coords(se0, se1):
    gx = 2.0 * ((se0 * 1024.0) / 2047.0 - 0.5)
    gy = 2.0 * ((se1 * 1024.0) / 1023.0 - 0.5)
    x = ((gx + 1.0) * 2048.0) / 2.0 - 0.5
    y = ((gy + 1.0) * 1024.0) / 2.0 - 0.5
    x0 = jnp.floor(x)
    y0 = jnp.floor(y)
    return x, y, x0, y0


def _clip_idx(xi, yi):
    xc = jnp.clip(xi, 0, W - 1).astype(jnp.int32)
    yc = jnp.clip(yi, 0, H - 1).astype(jnp.int32)
    return yc * W + xc


def _valid(xi, yi):
    return (xi >= 0) & (xi <= W - 1) & (yi >= 0) & (yi <= H - 1)


def _a_kernel(p0_r, p1_r, p5_r, p6_r, xm_r, ym_r, seed_o, se0_o, se1_o, idx_o):
    o0 = jnp.tanh(p0_r[...])
    o1 = jnp.tanh(p1_r[...])
    se0 = o0 + xm_r[...]
    se1 = o1 + ym_r[...]
    se0_o[...] = se0
    se1_o[...] = se1
    mx = jnp.maximum(p5_r[...], p6_r[...])
    e0 = jnp.exp(p5_r[...] - mx)
    e1 = jnp.exp(p6_r[...] - mx)
    seed_o[...] = e1 / (e0 + e1)
    x, y, x0, y0 = _coords(se0, se1)
    x1 = x0 + 1.0
    y1 = y0 + 1.0
    xc0 = jnp.clip(x0, 0, W - 1).astype(jnp.int32)
    xc1 = jnp.clip(x1, 0, W - 1).astype(jnp.int32)
    yc0 = jnp.clip(y0, 0, H - 1).astype(jnp.int32)
    yc1 = jnp.clip(y1, 0, H - 1).astype(jnp.int32)
    lin00 = yc0 * W + xc0
    lin20 = yc1 * W + xc0
    o0 = xc0 & 31
    x_any = (x0 >= -1.0) & (x0 <= jnp.float32(W - 1))
    row0_ok = x_any & (y0 >= 0.0) & (y0 <= jnp.float32(H - 1))
    row1_ok = x_any & (y1 >= 0.0) & (y1 <= jnp.float32(H - 1))
    idx_o[0] = jnp.where(row0_ok, lin00 >> 5, -1)
    idx_o[1] = jnp.where(row1_ok, lin20 >> 5, -1)
    idx_o[2] = o0
    idx_o[3] = o0 + (xc1 - xc0)


def _b_kernel(se0_r, se1_r, g0_r, g1_r, e0_o, e1_o):
    se0 = se0_r[...]
    se1 = se1_r[...]
    x, y, x0, y0 = _coords(se0, se1)
    x1 = x0 + 1.0
    y1 = y0 + 1.0
    wx1 = x - x0
    wx0 = 1.0 - wx1
    wy1 = y - y0
    wy0 = 1.0 - wy1
    ws = (wx0 * wy0, wx1 * wy0, wx0 * wy1, wx1 * wy1)
    vs = (_valid(x0, y0), _valid(x1, y0), _valid(x0, y1), _valid(x1, y1))
    acc0 = jnp.zeros_like(se0)
    acc1 = jnp.zeros_like(se1)
    for t in range(4):
        o0t = jnp.where(vs[t], jnp.tanh(g0_r[t]), 0.0)
        o1t = jnp.where(vs[t], jnp.tanh(g1_r[t]), 0.0)
        if t == 0:
            acc0 = o0t * ws[t]
            acc1 = o1t * ws[t]
        else:
            acc0 = acc0 + o0t * ws[t]
            acc1 = acc1 + o1t * ws[t]
    e0_o[...] = se0 + acc0
    e1_o[...] = se1 + acc1


def _c_kernel(seed_r, e0_r, e1_r, s0_r, s1_r, inst_o, uncl_s, prop_s):
    colid = jax.lax.broadcasted_iota(jnp.int32, (CH, W), 1)
    rowid = jax.lax.broadcasted_iota(jnp.int32, (CH, W), 0)

    def init_chunk(i, n):
        sl = pl.ds(i * CH, CH)
        mk = (seed_r[sl, :] > 0.5).astype(jnp.uint8)
        uncl_s[sl, :] = mk
        inst_o[sl, :] = jnp.zeros((CH, W), jnp.uint8)
        return n + jnp.sum(mk.astype(jnp.int32))

    n0 = jax.lax.fori_loop(0, NCH, init_chunk, jnp.int32(0))

    def body(carry):
        count, _un = carry

        def p12(i, c):
            m, idx = c
            sl = pl.ds(i * CH, CH)
            sc = jnp.where(uncl_s[sl, :].astype(jnp.int32) > 0,
                           seed_r[sl, :], 0.0)
            cm = jnp.max(sc)
            flat = (rowid + i * CH) * W + colid
            cidx = jnp.min(jnp.where(sc == cm, flat, HW))
            idx_new = jnp.where(cm > m, cidx, idx)
            return (jnp.maximum(m, cm), idx_new)

        _m, idx = jax.lax.fori_loop(0, NCH, p12, (jnp.float32(0.0), jnp.int32(HW)))
        r = idx // W
        c = idx % W

        def _gather(ref):
            row = ref[pl.ds(r, 1), :]
            return jnp.sum(jnp.where(colid[0:1, :] == c, row, 0.0))

        c0 = _gather(e0_r)
        c1 = _gather(e1_r)
        s0 = jnp.exp(_gather(s0_r) * 10.0)
        s1 = jnp.exp(_gather(s1_r) * 10.0)

        def p3(i, acc):
            psum, usum = acc
            sl = pl.ds(i * CH, CH)
            d0 = e0_r[sl, :] - c0
            d1 = e1_r[sl, :] - c1
            q = d0 * d0 * s0 + d1 * d1 * s1
            dist = jnp.exp(-1.0 * q)
            pr_i = ((dist > 0.5) & (seed_r[sl, :] > 0.5)).astype(jnp.int32)
            flat = (rowid + i * CH) * W + colid
            unc_i = jnp.where(flat == idx, 0,
                              uncl_s[sl, :].astype(jnp.int32))
            uncl_s[sl, :] = unc_i.astype(jnp.uint8)
            prop_s[sl, :] = pr_i.astype(jnp.uint8)
            psum = psum + jnp.sum(pr_i)
            usum = usum + jnp.sum(pr_i * unc_i)
            return (psum, usum)

        psum, usum = jax.lax.fori_loop(0, NCH, p3, (jnp.int32(0), jnp.int32(0)))
        ratio_ok = (usum.astype(jnp.float32)
                    / jnp.maximum(psum, 1).astype(jnp.float32)) > 0.5
        accept = (psum > 160) & ratio_ok
        acc_i = jnp.where(accept, jnp.int32(1), jnp.int32(0))
        lab_i = count & 255

        def p4(i, un):
            sl = pl.ds(i * CH, CH)
            pr_i = prop_s[sl, :].astype(jnp.int32)
            inst_i = inst_o[sl, :].astype(jnp.int32)
            inst_o[sl, :] = jnp.where(pr_i * acc_i > 0, lab_i,
                                      inst_i).astype(jnp.uint8)
            unc_i = jnp.where(pr_i > 0, 0, uncl_s[sl, :].astype(jnp.int32))
            uncl_s[sl, :] = unc_i.astype(jnp.uint8)
            return un + jnp.sum(unc_i)

        un_new = jax.lax.fori_loop(0, NCH, p4, jnp.int32(0))
        count_new = count + jnp.where(accept, jnp.int32(1), jnp.int32(0))
        return (count_new, un_new)

    count_fin, _ = jax.lax.while_loop(lambda cr: cr[1] > 160, body,
                                      (jnp.int32(1), n0))

    def rem(l, z):
        li = l & 255

        def cnt_chunk(i, n):
            sl = pl.ds(i * CH, CH)
            return n + jnp.sum(
                (inst_o[sl, :].astype(jnp.int32) == li).astype(jnp.int32))

        n = jax.lax.fori_loop(0, NCH, cnt_chunk, jnp.int32(0))

        @pl.when(n < 160)
        def _():
            def rm(i, zz):
                sl = pl.ds(i * CH, CH)
                inst_i = inst_o[sl, :].astype(jnp.int32)
                inst_o[sl, :] = jnp.where(inst_i == li, 0,
                                          inst_i).astype(jnp.uint8)
                return zz

            jax.lax.fori_loop(0, NCH, rm, jnp.int32(0))

        return z

    jax.lax.fori_loop(1, count_fin, rem, jnp.int32(0))


def _stage_a(p0, p1, p5, p6, xm, ym, interpret=False):
    f32 = jnp.float32
    return pl.pallas_call(
        _a_kernel,
        grid=(H // BR,),
        in_specs=[
            pl.BlockSpec((BR, W), lambda i: (i, 0)),
            pl.BlockSpec((BR, W), lambda i: (i, 0)),
            pl.BlockSpec((BR, W), lambda i: (i, 0)),
            pl.BlockSpec((BR, W), lambda i: (i, 0)),
            pl.BlockSpec((1, W), lambda i: (0, 0)),
            pl.BlockSpec((BR, 1), lambda i: (i, 0)),
        ],
        out_specs=[
            pl.BlockSpec((BR, W), lambda i: (i, 0)),
            pl.BlockSpec((BR, W), lambda i: (i, 0)),
            pl.BlockSpec((BR, W), lambda i: (i, 0)),
            pl.BlockSpec((4, BR, W), lambda i: (0, i, 0)),
        ],
        out_shape=[
            jax.ShapeDtypeStruct((H, W), f32),
            jax.ShapeDtypeStruct((H, W), f32),
            jax.ShapeDtypeStruct((H, W), f32),
            jax.ShapeDtypeStruct((4, H, W), jnp.int32),
        ],
        interpret=interpret,
    )(p0, p1, p5, p6, xm, ym)


def _stage_b(se0, se1, g0, g1, interpret=False):
    f32 = jnp.float32
    return pl.pallas_call(
        _b_kernel,
        grid=(H // BR,),
        in_specs=[
            pl.BlockSpec((BR, W), lambda i: (i, 0)),
            pl.BlockSpec((BR, W), lambda i: (i, 0)),
            pl.BlockSpec((4, BR, W), lambda i: (0, i, 0)),
            pl.BlockSpec((4, BR, W), lambda i: (0, i, 0)),
        ],
        out_specs=[
            pl.BlockSpec((BR, W), lambda i: (i, 0)),
            pl.BlockSpec((BR, W), lambda i: (i, 0)),
        ],
        out_shape=[
            jax.ShapeDtypeStruct((H, W), f32),
            jax.ShapeDtypeStruct((H, W), f32),
        ],
        interpret=interpret,
    )(se0, se1, g0, g1)


def _stage_c(seed, e0, e1, s0, s1, interpret=False):
    return pl.pallas_call(
        _c_kernel,
        in_specs=[pl.BlockSpec(memory_space=pltpu.VMEM)] * 5,
        out_specs=pl.BlockSpec(memory_space=pltpu.VMEM),
        out_shape=jax.ShapeDtypeStruct((H, W), jnp.uint8),
        scratch_shapes=[
            pltpu.VMEM((H, W), jnp.uint8),
            pltpu.VMEM((H, W), jnp.uint8),
        ],
        compiler_params=pltpu.CompilerParams(
            vmem_limit_bytes=100 * 1024 * 1024,
        ),
        interpret=interpret,
    )(seed, e0, e1, s0, s1)


SC_NW = 32
SC_PW = HW // SC_NW
SC_CHK = 256
SC_NIT = SC_PW // SC_CHK
NROW = HW // 32


def _mk_table(p0, p1):
    z = jnp.zeros((1, 32), jnp.float32)
    a0 = p0.reshape(NROW, 32)
    b0 = p1.reshape(NROW, 32)
    a1 = jnp.concatenate([a0[1:], z], axis=0)
    b1 = jnp.concatenate([b0[1:], z], axis=0)
    return jnp.concatenate([a0, a1, b0, b1], axis=1)


def _sc_gather(table, r0, r1, o0, o1):
    mesh = plsc.VectorSubcoreMesh(core_axis_name="c", subcore_axis_name="s")
    cp = pltpu.CompilerParams()
    if "needs_layout_passes" in pltpu.CompilerParams.__dataclass_fields__:
        cp = dataclasses.replace(cp, needs_layout_passes=False)

    @functools.partial(
        pl.kernel,
        mesh=mesh,
        compiler_params=cp,
        out_type=[
            jax.ShapeDtypeStruct((4, HW), jnp.float32),
            jax.ShapeDtypeStruct((4, HW), jnp.float32),
        ],
        scratch_types=[
            pltpu.VMEM((SC_CHK,), jnp.int32),
            pltpu.VMEM((SC_CHK,), jnp.int32),
            pltpu.VMEM((SC_CHK,), jnp.int32),
            pltpu.VMEM((SC_CHK,), jnp.int32),
            pltpu.VMEM((SC_CHK, 128), jnp.float32),
            pltpu.VMEM((SC_CHK, 128), jnp.float32),
            pltpu.VMEM((8, SC_CHK), jnp.float32),
            pltpu.SemaphoreType.DMA,
            pltpu.SemaphoreType.DMA,
        ],
    )
    def k(t_hbm, r0_hbm, r1_hbm, o0_hbm, o1_hbm, g0_hbm, g1_hbm,
          i0_v, i1_v, l0_v, l1_v, rows0_v, rows1_v, out_v, sem0, sem1):
        wid = lax.axis_index("s") * 2 + lax.axis_index("c")
        base = wid * SC_PW

        @pl.loop(0, SC_NIT)
        def _(it):
            off = base + it * SC_CHK
            pltpu.sync_copy(r0_hbm.at[pl.ds(off, SC_CHK)], i0_v)
            pltpu.sync_copy(r1_hbm.at[pl.ds(off, SC_CHK)], i1_v)
            cp0 = pltpu.async_copy(
                t_hbm.at[plsc.Indices(i0_v, ignored_value=-1)], rows0_v, sem0)
            cp1 = pltpu.async_copy(
                t_hbm.at[plsc.Indices(i1_v, ignored_value=-1)], rows1_v, sem1)
            pltpu.sync_copy(o0_hbm.at[pl.ds(off, SC_CHK)], l0_v)
            pltpu.sync_copy(o1_hbm.at[pl.ds(off, SC_CHK)], l1_v)
            cp0.wait()
            cp1.wait()

            @pl.loop(0, SC_CHK, step=16)
            def _(cb):
                rvec = cb + lax.iota(jnp.int32, 16)
                l0 = l0_v[pl.ds(cb, 16)]
                l1 = l1_v[pl.ds(cb, 16)]
                out_v[0, pl.ds(cb, 16)] = plsc.load_gather(rows0_v, [rvec, l0])
                out_v[1, pl.ds(cb, 16)] = plsc.load_gather(rows0_v, [rvec, l1])
                out_v[2, pl.ds(cb, 16)] = plsc.load_gather(rows1_v, [rvec, l0])
                out_v[3, pl.ds(cb, 16)] = plsc.load_gather(rows1_v, [rvec, l1])
                out_v[4, pl.ds(cb, 16)] = plsc.load_gather(rows0_v,
                                                           [rvec, l0 + 64])
                out_v[5, pl.ds(cb, 16)] = plsc.load_gather(rows0_v,
                                                           [rvec, l1 + 64])
                out_v[6, pl.ds(cb, 16)] = plsc.load_gather(rows1_v,
                                                           [rvec, l0 + 64])
                out_v[7, pl.ds(cb, 16)] = plsc.load_gather(rows1_v,
                                                           [rvec, l1 + 64])

            pltpu.sync_copy(out_v.at[0], g0_hbm.at[0, pl.ds(off, SC_CHK)])
            pltpu.sync_copy(out_v.at[1], g0_hbm.at[1, pl.ds(off, SC_CHK)])
            pltpu.sync_copy(out_v.at[2], g0_hbm.at[2, pl.ds(off, SC_CHK)])
            pltpu.sync_copy(out_v.at[3], g0_hbm.at[3, pl.ds(off, SC_CHK)])
            pltpu.sync_copy(out_v.at[4], g1_hbm.at[0, pl.ds(off, SC_CHK)])
            pltpu.sync_copy(out_v.at[5], g1_hbm.at[1, pl.ds(off, SC_CHK)])
            pltpu.sync_copy(out_v.at[6], g1_hbm.at[2, pl.ds(off, SC_CHK)])
            pltpu.sync_copy(out_v.at[7], g1_hbm.at[3, pl.ds(off, SC_CHK)])

    return k(table, r0, r1, o0, o1)


def _pipeline(prediction, interpret=False):
    pred = prediction[0]
    p0, p1 = pred[0], pred[1]
    sg0, sg1 = pred[2], pred[3]
    p5, p6 = pred[5], pred[6]
    xm = jnp.linspace(0.0, 2.0, 2048).reshape(1, W)
    ym = jnp.linspace(0.0, 1.0, 1024).reshape(H, 1)
    seed, se0, se1, idx4 = _stage_a(p0, p1, p5, p6, xm, ym, interpret=interpret)
    table = _mk_table(p0, p1)
    r0 = idx4[0].reshape(-1)
    r1 = idx4[1].reshape(-1)
    o0 = idx4[2].reshape(-1)
    o1 = idx4[3].reshape(-1)
    if interpret:
        g0f = jnp.stack([table[r0, o0], table[r0, o1],
                         table[r1, o0], table[r1, o1]])
        g1f = jnp.stack([table[r0, o0 + 64], table[r0, o1 + 64],
                         table[r1, o0 + 64], table[r1, o1 + 64]])
    else:
        g0f, g1f = _sc_gather(table, r0, r1, o0, o1)
    g0 = g0f.reshape(4, H, W)
    g1 = g1f.reshape(4, H, W)
    e0, e1 = _stage_b(se0, se1, g0, g1, interpret=interpret)
    inst = _stage_c(seed, e0, e1, sg0, sg1, interpret=interpret)
    return inst.reshape(1, H, W)


def kernel(prediction):
    return _pipeline(prediction)


# speedup vs baseline: 5.7274x; 1.0173x over previous
import dataclasses
import functools

import jax
import jax.numpy as jnp
from jax import lax
from jax.experimental import pallas as pl
from jax.experimental.pallas import tpu as pltpu
from jax.experimental.pallas import tpu_sc as plsc

H, W = 1024, 2048
HW = H * W
BR = 128
CH = 128
NCH = H // CH


def _coords(se0, se1):
    gx = 2.0 * ((se0 * 1024.0) / 2047.0 - 0.5)
    gy = 2.0 * ((se1 * 1024.0) / 1023.0 - 0.5)
    x = ((gx + 1.0) * 2048.0) / 2.0 - 0.5
    y = ((gy + 1.0) * 1024.0) / 2.0 - 0.5
    x0 = jnp.floor(x)
    y0 = jnp.floor(y)
    return x, y, x0, y0


def _clip_idx(xi, yi):
    xc = jnp.clip(xi, 0, W - 1).astype(jnp.int32)
    yc = jnp.clip(yi, 0, H - 1).astype(jnp.int32)
    return yc * W + xc


def _valid(xi, yi):
    return (xi >= 0) & (xi <= W - 1) & (yi >= 0) & (yi <= H - 1)


def _a_kernel(p0_r, p1_r, p5_r, p6_r, xm_r, ym_r, seed_o, se0_o, se1_o, idx_o):
    o0 = jnp.tanh(p0_r[...])
    o1 = jnp.tanh(p1_r[...])
    se0 = o0 + xm_r[...]
    se1 = o1 + ym_r[...]
    se0_o[...] = se0
    se1_o[...] = se1
    mx = jnp.maximum(p5_r[...], p6_r[...])
    e0 = jnp.exp(p5_r[...] - mx)
    e1 = jnp.exp(p6_r[...] - mx)
    sm = e1 / (e0 + e1)
    seed_o[...] = sm
    x, y, x0, y0 = _coords(se0, se1)
    x1 = x0 + 1.0
    y1 = y0 + 1.0
    xc0 = jnp.clip(x0, 0, W - 1).astype(jnp.int32)
    xc1 = jnp.clip(x1, 0, W - 1).astype(jnp.int32)
    yc0 = jnp.clip(y0, 0, H - 1).astype(jnp.int32)
    yc1 = jnp.clip(y1, 0, H - 1).astype(jnp.int32)
    lin00 = yc0 * W + xc0
    lin20 = yc1 * W + xc0
    o0 = xc0 & 31
    x_any = (x0 >= -1.0) & (x0 <= jnp.float32(W - 1)) & (sm > 0.5)
    row0_ok = x_any & (y0 >= 0.0) & (y0 <= jnp.float32(H - 1))
    row1_ok = x_any & (y1 >= 0.0) & (y1 <= jnp.float32(H - 1))
    idx_o[0] = jnp.where(row0_ok, lin00 >> 5, -1)
    idx_o[1] = jnp.where(row1_ok, lin20 >> 5, -1)
    idx_o[2] = o0
    idx_o[3] = o0 + (xc1 - xc0)


def _b_kernel(se0_r, se1_r, g0_r, g1_r, e0_o, e1_o):
    se0 = se0_r[...]
    se1 = se1_r[...]
    x, y, x0, y0 = _coords(se0, se1)
    x1 = x0 + 1.0
    y1 = y0 + 1.0
    wx1 = x - x0
    wx0 = 1.0 - wx1
    wy1 = y - y0
    wy0 = 1.0 - wy1
    ws = (wx0 * wy0, wx1 * wy0, wx0 * wy1, wx1 * wy1)
    vs = (_valid(x0, y0), _valid(x1, y0), _valid(x0, y1), _valid(x1, y1))
    acc0 = jnp.zeros_like(se0)
    acc1 = jnp.zeros_like(se1)
    for t in range(4):
        o0t = jnp.where(vs[t], jnp.tanh(g0_r[t]), 0.0)
        o1t = jnp.where(vs[t], jnp.tanh(g1_r[t]), 0.0)
        if t == 0:
            acc0 = o0t * ws[t]
            acc1 = o1t * ws[t]
        else:
            acc0 = acc0 + o0t * ws[t]
            acc1 = acc1 + o1t * ws[t]
    e0_o[...] = se0 + acc0
    e1_o[...] = se1 + acc1


def _c_kernel(seed_r, e0_r, e1_r, s0_r, s1_r, inst_o, uncl_s, prop_s):
    colid = jax.lax.broadcasted_iota(jnp.int32, (CH, W), 1)
    rowid = jax.lax.broadcasted_iota(jnp.int32, (CH, W), 0)

    def init_chunk(i, n):
        sl = pl.ds(i * CH, CH)
        mk = (seed_r[sl, :] > 0.5).astype(jnp.uint8)
        uncl_s[sl, :] = mk
        inst_o[sl, :] = jnp.zeros((CH, W), jnp.uint8)
        return n + jnp.sum(mk.astype(jnp.int32))

    n0 = jax.lax.fori_loop(0, NCH, init_chunk, jnp.int32(0))

    def body(carry):
        count, _un = carry

        def p12(i, c):
            m, idx = c
            sl = pl.ds(i * CH, CH)
            sc = jnp.where(uncl_s[sl, :].astype(jnp.int32) > 0,
                           seed_r[sl, :], 0.0)
            cm = jnp.max(sc)
            flat = (rowid + i * CH) * W + colid
            cidx = jnp.min(jnp.where(sc == cm, flat, HW))
            idx_new = jnp.where(cm > m, cidx, idx)
            return (jnp.maximum(m, cm), idx_new)

        _m, idx = jax.lax.fori_loop(0, NCH, p12, (jnp.float32(0.0), jnp.int32(HW)))
        r = idx // W
        c = idx % W

        def _gather(ref):
            row = ref[pl.ds(r, 1), :]
            return jnp.sum(jnp.where(colid[0:1, :] == c, row, 0.0))

        c0 = _gather(e0_r)
        c1 = _gather(e1_r)
        s0 = jnp.exp(_gather(s0_r) * 10.0)
        s1 = jnp.exp(_gather(s1_r) * 10.0)

        def p3(i, acc):
            psum, usum = acc
            sl = pl.ds(i * CH, CH)
            d0 = e0_r[sl, :] - c0
            d1 = e1_r[sl, :] - c1
            q = d0 * d0 * s0 + d1 * d1 * s1
            dist = jnp.exp(-1.0 * q)
            pr_i = ((dist > 0.5) & (seed_r[sl, :] > 0.5)).astype(jnp.int32)
            flat = (rowid + i * CH) * W + colid
            unc_i = jnp.where(flat == idx, 0,
                              uncl_s[sl, :].astype(jnp.int32))
            uncl_s[sl, :] = unc_i.astype(jnp.uint8)
            prop_s[sl, :] = pr_i.astype(jnp.uint8)
            psum = psum + jnp.sum(pr_i)
            usum = usum + jnp.sum(pr_i * unc_i)
            return (psum, usum)

        psum, usum = jax.lax.fori_loop(0, NCH, p3, (jnp.int32(0), jnp.int32(0)))
        ratio_ok = (usum.astype(jnp.float32)
                    / jnp.maximum(psum, 1).astype(jnp.float32)) > 0.5
        accept = (psum > 160) & ratio_ok
        acc_i = jnp.where(accept, jnp.int32(1), jnp.int32(0))
        lab_i = count & 255

        def p4(i, un):
            sl = pl.ds(i * CH, CH)
            pr_i = prop_s[sl, :].astype(jnp.int32)
            inst_i = inst_o[sl, :].astype(jnp.int32)
            inst_o[sl, :] = jnp.where(pr_i * acc_i > 0, lab_i,
                                      inst_i).astype(jnp.uint8)
            unc_i = jnp.where(pr_i > 0, 0, uncl_s[sl, :].astype(jnp.int32))
            uncl_s[sl, :] = unc_i.astype(jnp.uint8)
            return un + jnp.sum(unc_i)

        un_new = jax.lax.fori_loop(0, NCH, p4, jnp.int32(0))
        count_new = count + jnp.where(accept, jnp.int32(1), jnp.int32(0))
        return (count_new, un_new)

    count_fin, _ = jax.lax.while_loop(lambda cr: cr[1] > 160, body,
                                      (jnp.int32(1), n0))

    def rem(l, z):
        li = l & 255

        def cnt_chunk(i, n):
            sl = pl.ds(i * CH, CH)
            return n + jnp.sum(
                (inst_o[sl, :].astype(jnp.int32) == li).astype(jnp.int32))

        n = jax.lax.fori_loop(0, NCH, cnt_chunk, jnp.int32(0))

        @pl.when(n < 160)
        def _():
            def rm(i, zz):
                sl = pl.ds(i * CH, CH)
                inst_i = inst_o[sl, :].astype(jnp.int32)
                inst_o[sl, :] = jnp.where(inst_i == li, 0,
                                          inst_i).astype(jnp.uint8)
                return zz

            jax.lax.fori_loop(0, NCH, rm, jnp.int32(0))

        return z

    jax.lax.fori_loop(1, count_fin, rem, jnp.int32(0))


def _stage_a(p0, p1, p5, p6, xm, ym, interpret=False):
    f32 = jnp.float32
    return pl.pallas_call(
        _a_kernel,
        grid=(H // BR,),
        in_specs=[
            pl.BlockSpec((BR, W), lambda i: (i, 0)),
            pl.BlockSpec((BR, W), lambda i: (i, 0)),
            pl.BlockSpec((BR, W), lambda i: (i, 0)),
            pl.BlockSpec((BR, W), lambda i: (i, 0)),
            pl.BlockSpec((1, W), lambda i: (0, 0)),
            pl.BlockSpec((BR, 1), lambda i: (i, 0)),
        ],
        out_specs=[
            pl.BlockSpec((BR, W), lambda i: (i, 0)),
            pl.BlockSpec((BR, W), lambda i: (i, 0)),
            pl.BlockSpec((BR, W), lambda i: (i, 0)),
            pl.BlockSpec((4, BR, W), lambda i: (0, i, 0)),
        ],
        out_shape=[
            jax.ShapeDtypeStruct((H, W), f32),
            jax.ShapeDtypeStruct((H, W), f32),
            jax.ShapeDtypeStruct((H, W), f32),
            jax.ShapeDtypeStruct((4, H, W), jnp.int32),
        ],
        interpret=interpret,
    )(p0, p1, p5, p6, xm, ym)


def _stage_b(se0, se1, g0, g1, interpret=False):
    f32 = jnp.float32
    return pl.pallas_call(
        _b_kernel,
        grid=(H // BR,),
        in_specs=[
            pl.BlockSpec((BR, W), lambda i: (i, 0)),
            pl.BlockSpec((BR, W), lambda i: (i, 0)),
            pl.BlockSpec((4, BR, W), lambda i: (0, i, 0)),
            pl.BlockSpec((4, BR, W), lambda i: (0, i, 0)),
        ],
        out_specs=[
            pl.BlockSpec((BR, W), lambda i: (i, 0)),
            pl.BlockSpec((BR, W), lambda i: (i, 0)),
        ],
        out_shape=[
            jax.ShapeDtypeStruct((H, W), f32),
            jax.ShapeDtypeStruct((H, W), f32),
        ],
        interpret=interpret,
    )(se0, se1, g0, g1)


def _stage_c(seed, e0, e1, s0, s1, interpret=False):
    return pl.pallas_call(
        _c_kernel,
        in_specs=[pl.BlockSpec(memory_space=pltpu.VMEM)] * 5,
        out_specs=pl.BlockSpec(memory_space=pltpu.VMEM),
        out_shape=jax.ShapeDtypeStruct((H, W), jnp.uint8),
        scratch_shapes=[
            pltpu.VMEM((H, W), jnp.uint8),
            pltpu.VMEM((H, W), jnp.uint8),
        ],
        compiler_params=pltpu.CompilerParams(
            vmem_limit_bytes=100 * 1024 * 1024,
        ),
        interpret=interpret,
    )(seed, e0, e1, s0, s1)


SC_NW = 32
SC_PW = HW // SC_NW
SC_CHK = 256
SC_NIT = SC_PW // SC_CHK
NROW = HW // 32


def _mk_table(p0, p1):
    z = jnp.zeros((1, 32), jnp.float32)
    a0 = p0.reshape(NROW, 32)
    b0 = p1.reshape(NROW, 32)
    a1 = jnp.concatenate([a0[1:], z], axis=0)
    b1 = jnp.concatenate([b0[1:], z], axis=0)
    return jnp.concatenate([a0, a1, b0, b1], axis=1)


def _sc_gather(table, r0, r1, o0, o1):
    mesh = plsc.VectorSubcoreMesh(core_axis_name="c", subcore_axis_name="s")
    cp = pltpu.CompilerParams()
    if "needs_layout_passes" in pltpu.CompilerParams.__dataclass_fields__:
        cp = dataclasses.replace(cp, needs_layout_passes=False)

    @functools.partial(
        pl.kernel,
        mesh=mesh,
        compiler_params=cp,
        out_type=[
            jax.ShapeDtypeStruct((4, HW), jnp.float32),
            jax.ShapeDtypeStruct((4, HW), jnp.float32),
        ],
        scratch_types=[
            pltpu.VMEM((SC_CHK,), jnp.int32),
            pltpu.VMEM((SC_CHK,), jnp.int32),
            pltpu.VMEM((SC_CHK,), jnp.int32),
            pltpu.VMEM((SC_CHK,), jnp.int32),
            pltpu.VMEM((SC_CHK, 128), jnp.float32),
            pltpu.VMEM((SC_CHK, 128), jnp.float32),
            pltpu.VMEM((8, SC_CHK), jnp.float32),
            pltpu.SemaphoreType.DMA,
            pltpu.SemaphoreType.DMA,
        ],
    )
    def k(t_hbm, r0_hbm, r1_hbm, o0_hbm, o1_hbm, g0_hbm, g1_hbm,
          i0_v, i1_v, l0_v, l1_v, rows0_v, rows1_v, out_v, sem0, sem1):
        wid = lax.axis_index("s") * 2 + lax.axis_index("c")
        base = wid * SC_PW

        @pl.loop(0, SC_NIT)
        def _(it):
            off = base + it * SC_CHK
            pltpu.sync_copy(r0_hbm.at[pl.ds(off, SC_CHK)], i0_v)
            pltpu.sync_copy(r1_hbm.at[pl.ds(off, SC_CHK)], i1_v)
            cp0 = pltpu.async_copy(
                t_hbm.at[plsc.Indices(i0_v, ignored_value=-1)], rows0_v, sem0)
            cp1 = pltpu.async_copy(
                t_hbm.at[plsc.Indices(i1_v, ignored_value=-1)], rows1_v, sem1)
            pltpu.sync_copy(o0_hbm.at[pl.ds(off, SC_CHK)], l0_v)
            pltpu.sync_copy(o1_hbm.at[pl.ds(off, SC_CHK)], l1_v)
            cp0.wait()
            cp1.wait()

            @pl.loop(0, SC_CHK, step=16)
            def _(cb):
                rvec = cb + lax.iota(jnp.int32, 16)
                l0 = l0_v[pl.ds(cb, 16)]
                l1 = l1_v[pl.ds(cb, 16)]
                out_v[0, pl.ds(cb, 16)] = plsc.load_gather(rows0_v, [rvec, l0])
                out_v[1, pl.ds(cb, 16)] = plsc.load_gather(rows0_v, [rvec, l1])
                out_v[2, pl.ds(cb, 16)] = plsc.load_gather(rows1_v, [rvec, l0])
                out_v[3, pl.ds(cb, 16)] = plsc.load_gather(rows1_v, [rvec, l1])
                out_v[4, pl.ds(cb, 16)] = plsc.load_gather(rows0_v,
                                                           [rvec, l0 + 64])
                out_v[5, pl.ds(cb, 16)] = plsc.load_gather(rows0_v,
                                                           [rvec, l1 + 64])
                out_v[6, pl.ds(cb, 16)] = plsc.load_gather(rows1_v,
                                                           [rvec, l0 + 64])
                out_v[7, pl.ds(cb, 16)] = plsc.load_gather(rows1_v,
                                                           [rvec, l1 + 64])

            pltpu.sync_copy(out_v.at[0], g0_hbm.at[0, pl.ds(off, SC_CHK)])
            pltpu.sync_copy(out_v.at[1], g0_hbm.at[1, pl.ds(off, SC_CHK)])
            pltpu.sync_copy(out_v.at[2], g0_hbm.at[2, pl.ds(off, SC_CHK)])
            pltpu.sync_copy(out_v.at[3], g0_hbm.at[3, pl.ds(off, SC_CHK)])
            pltpu.sync_copy(out_v.at[4], g1_hbm.at[0, pl.ds(off, SC_CHK)])
            pltpu.sync_copy(out_v.at[5], g1_hbm.at[1, pl.ds(off, SC_CHK)])
            pltpu.sync_copy(out_v.at[6], g1_hbm.at[2, pl.ds(off, SC_CHK)])
            pltpu.sync_copy(out_v.at[7], g1_hbm.at[3, pl.ds(off, SC_CHK)])

    return k(table, r0, r1, o0, o1)


def _pipeline(prediction, interpret=False):
    pred = prediction[0]
    p0, p1 = pred[0], pred[1]
    sg0, sg1 = pred[2], pred[3]
    p5, p6 = pred[5], pred[6]
    xm = jnp.linspace(0.0, 2.0, 2048).reshape(1, W)
    ym = jnp.linspace(0.0, 1.0, 1024).reshape(H, 1)
    seed, se0, se1, idx4 = _stage_a(p0, p1, p5, p6, xm, ym, interpret=interpret)
    table = _mk_table(p0, p1)
    r0 = idx4[0].reshape(-1)
    r1 = idx4[1].reshape(-1)
    o0 = idx4[2].reshape(-1)
    o1 = idx4[3].reshape(-1)
    if interpret:
        g0f = jnp.stack([table[r0, o0], table[r0, o1],
                         table[r1, o0], table[r1, o1]])
        g1f = jnp.stack([table[r0, o0 + 64], table[r0, o1 + 64],
                         table[r1, o0 + 64], table[r1, o1 + 64]])
    else:
        g0f, g1f = _sc_gather(table, r0, r1, o0, o1)
    g0 = g0f.reshape(4, H, W)
    g1 = g1f.reshape(4, H, W)
    e0, e1 = _stage_b(se0, se1, g0, g1, interpret=interpret)
    inst = _stage_c(seed, e0, e1, sg0, sg1, interpret=interpret)
    return inst.reshape(1, H, W)


def kernel(prediction):
    return _pipeline(prediction)


# speedup vs baseline: 10.3834x; 1.8129x over previous
import dataclasses
import functools

import jax
import jax.numpy as jnp
from jax import lax
from jax.experimental import pallas as pl
from jax.experimental.pallas import tpu as pltpu
from jax.experimental.pallas import tpu_sc as plsc

H, W = 1024, 2048
HW = H * W
BR = 128
CH = 128
NCH = H // CH


def _coords(se0, se1):
    gx = 2.0 * ((se0 * 1024.0) / 2047.0 - 0.5)
    gy = 2.0 * ((se1 * 1024.0) / 1023.0 - 0.5)
    x = ((gx + 1.0) * 2048.0) / 2.0 - 0.5
    y = ((gy + 1.0) * 1024.0) / 2.0 - 0.5
    x0 = jnp.floor(x)
    y0 = jnp.floor(y)
    return x, y, x0, y0


def _clip_idx(xi, yi):
    xc = jnp.clip(xi, 0, W - 1).astype(jnp.int32)
    yc = jnp.clip(yi, 0, H - 1).astype(jnp.int32)
    return yc * W + xc


def _valid(xi, yi):
    return (xi >= 0) & (xi <= W - 1) & (yi >= 0) & (yi <= H - 1)


def _a_kernel(p0_r, p1_r, p5_r, p6_r, xm_r, ym_r, seed_o, se0_o, se1_o, idx_o):
    o0 = jnp.tanh(p0_r[...])
    o1 = jnp.tanh(p1_r[...])
    se0 = o0 + xm_r[...]
    se1 = o1 + ym_r[...]
    se0_o[...] = se0
    se1_o[...] = se1
    mx = jnp.maximum(p5_r[...], p6_r[...])
    e0 = jnp.exp(p5_r[...] - mx)
    e1 = jnp.exp(p6_r[...] - mx)
    sm = e1 / (e0 + e1)
    seed_o[...] = sm
    x, y, x0, y0 = _coords(se0, se1)
    x1 = x0 + 1.0
    y1 = y0 + 1.0
    xc0 = jnp.clip(x0, 0, W - 1).astype(jnp.int32)
    xc1 = jnp.clip(x1, 0, W - 1).astype(jnp.int32)
    yc0 = jnp.clip(y0, 0, H - 1).astype(jnp.int32)
    yc1 = jnp.clip(y1, 0, H - 1).astype(jnp.int32)
    lin00 = yc0 * W + xc0
    lin20 = yc1 * W + xc0
    o0 = xc0 & 31
    x_any = (x0 >= -1.0) & (x0 <= jnp.float32(W - 1)) & (sm > 0.5)
    row0_ok = x_any & (y0 >= 0.0) & (y0 <= jnp.float32(H - 1))
    row1_ok = x_any & (y1 >= 0.0) & (y1 <= jnp.float32(H - 1))
    idx_o[...] = jnp.stack(
        [jnp.where(row0_ok, lin00 >> 5, -1),
         jnp.where(row1_ok, lin20 >> 5, -1),
         o0,
         o0 + (xc1 - xc0)], axis=1)


def _b_kernel(se0_r, se1_r, g_r, e0_o, e1_o):
    se0 = se0_r[...]
    se1 = se1_r[...]
    x, y, x0, y0 = _coords(se0, se1)
    x1 = x0 + 1.0
    y1 = y0 + 1.0
    wx1 = x - x0
    wx0 = 1.0 - wx1
    wy1 = y - y0
    wy0 = 1.0 - wy1
    ws = (wx0 * wy0, wx1 * wy0, wx0 * wy1, wx1 * wy1)
    vs = (_valid(x0, y0), _valid(x1, y0), _valid(x0, y1), _valid(x1, y1))
    acc0 = jnp.zeros_like(se0)
    acc1 = jnp.zeros_like(se1)
    for t in range(4):
        o0t = jnp.where(vs[t], jnp.tanh(g_r[:, t, :]), 0.0)
        o1t = jnp.where(vs[t], jnp.tanh(g_r[:, 4 + t, :]), 0.0)
        if t == 0:
            acc0 = o0t * ws[t]
            acc1 = o1t * ws[t]
        else:
            acc0 = acc0 + o0t * ws[t]
            acc1 = acc1 + o1t * ws[t]
    e0_o[...] = se0 + acc0
    e1_o[...] = se1 + acc1


def _c_kernel(seed_r, e0_r, e1_r, s0_r, s1_r, inst_o, uncl_s, prop_s):
    colid = jax.lax.broadcasted_iota(jnp.int32, (CH, W), 1)
    rowid = jax.lax.broadcasted_iota(jnp.int32, (CH, W), 0)

    def init_chunk(i, n):
        sl = pl.ds(i * CH, CH)
        mk = (seed_r[sl, :] > 0.5).astype(jnp.uint8)
        uncl_s[sl, :] = mk
        inst_o[sl, :] = jnp.zeros((CH, W), jnp.uint8)
        return n + jnp.sum(mk.astype(jnp.int32))

    n0 = jax.lax.fori_loop(0, NCH, init_chunk, jnp.int32(0))

    def body(carry):
        count, _un = carry

        def p12(i, c):
            m, idx = c
            sl = pl.ds(i * CH, CH)
            sc = jnp.where(uncl_s[sl, :].astype(jnp.int32) > 0,
                           seed_r[sl, :], 0.0)
            cm = jnp.max(sc)
            flat = (rowid + i * CH) * W + colid
            cidx = jnp.min(jnp.where(sc == cm, flat, HW))
            idx_new = jnp.where(cm > m, cidx, idx)
            return (jnp.maximum(m, cm), idx_new)

        _m, idx = jax.lax.fori_loop(0, NCH, p12, (jnp.float32(0.0), jnp.int32(HW)))
        r = idx // W
        c = idx % W

        def _gather(ref):
            row = ref[pl.ds(r, 1), :]
            return jnp.sum(jnp.where(colid[0:1, :] == c, row, 0.0))

        c0 = _gather(e0_r)
        c1 = _gather(e1_r)
        s0 = jnp.exp(_gather(s0_r) * 10.0)
        s1 = jnp.exp(_gather(s1_r) * 10.0)

        def p3(i, acc):
            psum, usum = acc
            sl = pl.ds(i * CH, CH)
            d0 = e0_r[sl, :] - c0
            d1 = e1_r[sl, :] - c1
            q = d0 * d0 * s0 + d1 * d1 * s1
            dist = jnp.exp(-1.0 * q)
            pr_i = ((dist > 0.5) & (seed_r[sl, :] > 0.5)).astype(jnp.int32)
            flat = (rowid + i * CH) * W + colid
            unc_i = jnp.where(flat == idx, 0,
                              uncl_s[sl, :].astype(jnp.int32))
            uncl_s[sl, :] = unc_i.astype(jnp.uint8)
            prop_s[sl, :] = pr_i.astype(jnp.uint8)
            psum = psum + jnp.sum(pr_i)
            usum = usum + jnp.sum(pr_i * unc_i)
            return (psum, usum)

        psum, usum = jax.lax.fori_loop(0, NCH, p3, (jnp.int32(0), jnp.int32(0)))
        ratio_ok = (usum.astype(jnp.float32)
                    / jnp.maximum(psum, 1).astype(jnp.float32)) > 0.5
        accept = (psum > 160) & ratio_ok
        acc_i = jnp.where(accept, jnp.int32(1), jnp.int32(0))
        lab_i = count & 255

        def p4(i, un):
            sl = pl.ds(i * CH, CH)
            pr_i = prop_s[sl, :].astype(jnp.int32)
            inst_i = inst_o[sl, :].astype(jnp.int32)
            inst_o[sl, :] = jnp.where(pr_i * acc_i > 0, lab_i,
                                      inst_i).astype(jnp.uint8)
            unc_i = jnp.where(pr_i > 0, 0, uncl_s[sl, :].astype(jnp.int32))
            uncl_s[sl, :] = unc_i.astype(jnp.uint8)
            return un + jnp.sum(unc_i)

        un_new = jax.lax.fori_loop(0, NCH, p4, jnp.int32(0))
        count_new = count + jnp.where(accept, jnp.int32(1), jnp.int32(0))
        return (count_new, un_new)

    count_fin, _ = jax.lax.while_loop(lambda cr: cr[1] > 160, body,
                                      (jnp.int32(1), n0))

    def rem(l, z):
        li = l & 255

        def cnt_chunk(i, n):
            sl = pl.ds(i * CH, CH)
            return n + jnp.sum(
                (inst_o[sl, :].astype(jnp.int32) == li).astype(jnp.int32))

        n = jax.lax.fori_loop(0, NCH, cnt_chunk, jnp.int32(0))

        @pl.when(n < 160)
        def _():
            def rm(i, zz):
                sl = pl.ds(i * CH, CH)
                inst_i = inst_o[sl, :].astype(jnp.int32)
                inst_o[sl, :] = jnp.where(inst_i == li, 0,
                                          inst_i).astype(jnp.uint8)
                return zz

            jax.lax.fori_loop(0, NCH, rm, jnp.int32(0))

        return z

    jax.lax.fori_loop(1, count_fin, rem, jnp.int32(0))


def _stage_a(p0, p1, p5, p6, xm, ym, interpret=False):
    f32 = jnp.float32
    return pl.pallas_call(
        _a_kernel,
        grid=(H // BR,),
        in_specs=[
            pl.BlockSpec((BR, W), lambda i: (i, 0)),
            pl.BlockSpec((BR, W), lambda i: (i, 0)),
            pl.BlockSpec((BR, W), lambda i: (i, 0)),
            pl.BlockSpec((BR, W), lambda i: (i, 0)),
            pl.BlockSpec((1, W), lambda i: (0, 0)),
            pl.BlockSpec((BR, 1), lambda i: (i, 0)),
        ],
        out_specs=[
            pl.BlockSpec((BR, W), lambda i: (i, 0)),
            pl.BlockSpec((BR, W), lambda i: (i, 0)),
            pl.BlockSpec((BR, W), lambda i: (i, 0)),
            pl.BlockSpec((BR, 4, W), lambda i: (i, 0, 0)),
        ],
        out_shape=[
            jax.ShapeDtypeStruct((H, W), f32),
            jax.ShapeDtypeStruct((H, W), f32),
            jax.ShapeDtypeStruct((H, W), f32),
            jax.ShapeDtypeStruct((H, 4, W), jnp.int32),
        ],
        interpret=interpret,
    )(p0, p1, p5, p6, xm, ym)


def _stage_b(se0, se1, g, interpret=False):
    f32 = jnp.float32
    return pl.pallas_call(
        _b_kernel,
        grid=(H // BR,),
        in_specs=[
            pl.BlockSpec((BR, W), lambda i: (i, 0)),
            pl.BlockSpec((BR, W), lambda i: (i, 0)),
            pl.BlockSpec((BR, 8, W), lambda i: (i, 0, 0)),
        ],
        out_specs=[
            pl.BlockSpec((BR, W), lambda i: (i, 0)),
            pl.BlockSpec((BR, W), lambda i: (i, 0)),
        ],
        out_shape=[
            jax.ShapeDtypeStruct((H, W), f32),
            jax.ShapeDtypeStruct((H, W), f32),
        ],
        interpret=interpret,
    )(se0, se1, g)


def _stage_c(seed, e0, e1, s0, s1, interpret=False):
    return pl.pallas_call(
        _c_kernel,
        in_specs=[pl.BlockSpec(memory_space=pltpu.VMEM)] * 5,
        out_specs=pl.BlockSpec(memory_space=pltpu.VMEM),
        out_shape=jax.ShapeDtypeStruct((H, W), jnp.uint8),
        scratch_shapes=[
            pltpu.VMEM((H, W), jnp.uint8),
            pltpu.VMEM((H, W), jnp.uint8),
        ],
        compiler_params=pltpu.CompilerParams(
            vmem_limit_bytes=100 * 1024 * 1024,
        ),
        interpret=interpret,
    )(seed, e0, e1, s0, s1)


SC_NW = 32
SC_ROWS = H // SC_NW
SC_CHK = 128
SC_NCHK = W // SC_CHK
NROW = HW // 32


def _mk_table(p0, p1):
    z = jnp.zeros((1, 32), jnp.float32)
    a0 = p0.reshape(NROW, 32)
    b0 = p1.reshape(NROW, 32)
    a1 = jnp.concatenate([a0[1:], z], axis=0)
    b1 = jnp.concatenate([b0[1:], z], axis=0)
    return jnp.concatenate([a0, a1, b0, b1], axis=1)


def _sc_gather(table, rlo):
    mesh = plsc.VectorSubcoreMesh(core_axis_name="c", subcore_axis_name="s")
    cp = pltpu.CompilerParams()
    if "needs_layout_passes" in pltpu.CompilerParams.__dataclass_fields__:
        cp = dataclasses.replace(cp, needs_layout_passes=False)

    @functools.partial(
        pl.kernel,
        mesh=mesh,
        compiler_params=cp,
        out_type=jax.ShapeDtypeStruct((H, 8, W), jnp.float32),
        scratch_types=[
            pltpu.VMEM((2, 4, W), jnp.int32),
            pltpu.VMEM((2, SC_CHK, 128), jnp.float32),
            pltpu.VMEM((2, SC_CHK, 128), jnp.float32),
            pltpu.VMEM((2, 8, W), jnp.float32),
            pltpu.SemaphoreType.DMA,
            pltpu.SemaphoreType.DMA,
            pltpu.SemaphoreType.DMA,
            pltpu.SemaphoreType.DMA,
            pltpu.SemaphoreType.DMA,
            pltpu.SemaphoreType.DMA,
            pltpu.SemaphoreType.DMA,
            pltpu.SemaphoreType.DMA,
        ],
    )
    def k(t_hbm, rlo_hbm, g_hbm, ri_v, rows0_v, rows1_v, out_v,
          sem_ri0, sem_ri1, sem_r0a, sem_r0b, sem_r1a, sem_r1b,
          sem_o0, sem_o1):
        wid = lax.axis_index("s") * 2 + lax.axis_index("c")
        row0 = wid * SC_ROWS
        sem_ri = (sem_ri0, sem_ri1)
        sem_r0 = (sem_r0a, sem_r0b)
        sem_r1 = (sem_r1a, sem_r1b)
        sem_o = (sem_o0, sem_o1)

        def ri_copy(sb, b):
            return pltpu.make_async_copy(rlo_hbm.at[row0 + sb], ri_v.at[b],
                                         sem_ri[b])

        def out_copy(sb, b):
            return pltpu.make_async_copy(out_v.at[b], g_hbm.at[row0 + sb],
                                         sem_o[b])

        ri_copy(0, 0).start()
        ri_copy(1, 1).start()

        @pl.loop(0, SC_ROWS, step=2)
        def _(sb0):
            for b in range(2):
                sb = sb0 + b
                ri_copy(sb, b).wait()

                @pl.when(sb >= 2)
                def _():
                    out_copy(sb, b).wait()

                def stream(c, rb):
                    i0 = ri_v.at[b, 0, pl.ds(c * SC_CHK, SC_CHK)]
                    i1 = ri_v.at[b, 1, pl.ds(c * SC_CHK, SC_CHK)]
                    c0 = pltpu.make_async_copy(
                        t_hbm.at[plsc.Indices(i0, ignored_value=-1)],
                        rows0_v.at[rb], sem_r0[rb])
                    c1 = pltpu.make_async_copy(
                        t_hbm.at[plsc.Indices(i1, ignored_value=-1)],
                        rows1_v.at[rb], sem_r1[rb])
                    c0.start()
                    c1.start()
                    return (c0, c1)

                pending = [stream(0, 0), stream(1, 1)]
                for c in range(SC_NCHK):
                    rb = c & 1
                    c0, c1 = pending[rb]
                    c0.wait()
                    c1.wait()

                    @pl.loop(0, SC_CHK, step=16)
                    def _(cb):
                        rvec = cb + lax.iota(jnp.int32, 16)
                        pos = c * SC_CHK + cb
                        l0 = ri_v[b, 2, pl.ds(pos, 16)]
                        l1 = ri_v[b, 3, pl.ds(pos, 16)]
                        r0v = rows0_v.at[rb]
                        r1v = rows1_v.at[rb]
                        out_v[b, 0, pl.ds(pos, 16)] = plsc.load_gather(
                            r0v, [rvec, l0])
                        out_v[b, 1, pl.ds(pos, 16)] = plsc.load_gather(
                            r0v, [rvec, l1])
                        out_v[b, 2, pl.ds(pos, 16)] = plsc.load_gather(
                            r1v, [rvec, l0])
                        out_v[b, 3, pl.ds(pos, 16)] = plsc.load_gather(
                            r1v, [rvec, l1])
                        out_v[b, 4, pl.ds(pos, 16)] = plsc.load_gather(
                            r0v, [rvec, l0 + 64])
                        out_v[b, 5, pl.ds(pos, 16)] = plsc.load_gather(
                            r0v, [rvec, l1 + 64])
                        out_v[b, 6, pl.ds(pos, 16)] = plsc.load_gather(
                            r1v, [rvec, l0 + 64])
                        out_v[b, 7, pl.ds(pos, 16)] = plsc.load_gather(
                            r1v, [rvec, l1 + 64])

                    if c + 2 < SC_NCHK:
                        pending[rb] = stream(c + 2, rb)

                out_copy(sb, b).start()

                @pl.when(sb + 2 < SC_ROWS)
                def _():
                    ri_copy(sb + 2, b).start()

        out_copy(SC_ROWS - 2, 0).wait()
        out_copy(SC_ROWS - 1, 1).wait()

    return k(table, rlo)


def _pipeline(prediction, interpret=False):
    pred = prediction[0]
    p0, p1 = pred[0], pred[1]
    sg0, sg1 = pred[2], pred[3]
    p5, p6 = pred[5], pred[6]
    xm = jnp.linspace(0.0, 2.0, 2048).reshape(1, W)
    ym = jnp.linspace(0.0, 1.0, 1024).reshape(H, 1)
    seed, se0, se1, rlo = _stage_a(p0, p1, p5, p6, xm, ym, interpret=interpret)
    table = _mk_table(p0, p1)
    if interpret:
        r0 = rlo[:, 0, :].reshape(-1)
        r1 = rlo[:, 1, :].reshape(-1)
        o0 = rlo[:, 2, :].reshape(-1)
        o1 = rlo[:, 3, :].reshape(-1)
        g = jnp.stack([table[r0, o0], table[r0, o1],
                       table[r1, o0], table[r1, o1],
                       table[r0, o0 + 64], table[r0, o1 + 64],
                       table[r1, o0 + 64], table[r1, o1 + 64]])
        g = g.reshape(8, H, W).transpose(1, 0, 2)
    else:
        g = _sc_gather(table, rlo)
    e0, e1 = _stage_b(se0, se1, g, interpret=interpret)
    inst = _stage_c(seed, e0, e1, sg0, sg1, interpret=interpret)
    return inst.reshape(1, H, W)


def kernel(prediction):
    return _pipeline(prediction)
